# baseline (speedup 1.0000x reference)
import jax
import jax.numpy as jnp
from jax.experimental import pallas as pl
from jax.experimental.pallas import tpu as pltpu

HEADS = 32
EPS = 1e-5

TM_A = 512
KT_A = 1024
TM_B = 1024
TN_B = 1024


def _fc1_ln_kernel(x_ref, w1_ref, gamma_ref, beta_ref, w2h_ref,
                   h_ref, heads_ref, acc_ref):
    k = pl.program_id(1)
    nk = pl.num_programs(1)

    @pl.when(k == 0)
    def _():
        acc_ref[...] = jnp.zeros_like(acc_ref)

    acc_ref[...] += jnp.dot(x_ref[...].astype(jnp.bfloat16), w1_ref[...],
                            preferred_element_type=jnp.float32)

    @pl.when(k == nk - 1)
    def _():
        h = acc_ref[...]
        mu = jnp.mean(h, axis=-1, keepdims=True)
        var = jnp.mean((h - mu) ** 2, axis=-1, keepdims=True)
        hn = (h - mu) * jax.lax.rsqrt(var + EPS) * gamma_ref[...] + beta_ref[...]
        hnb = hn.astype(jnp.bfloat16)
        h_ref[...] = hnb
        heads_ref[...] = jnp.dot(hnb, w2h_ref[...],
                                 preferred_element_type=jnp.float32)


def _fc2_kernel(h_ref, w2n_ref, out_ref):
    out_ref[...] = jnp.dot(h_ref[...], w2n_ref[...],
                           preferred_element_type=jnp.float32)


def kernel(x, W1, gamma, beta, W2):
    n_tokens, embed = x.shape
    hidden = W1.shape[0]
    n_out = W2.shape[0]
    n_neurons = n_out - HEADS

    W1T = W1.T.astype(jnp.bfloat16)
    W2nT = W2[:n_neurons, :].T.astype(jnp.bfloat16)
    W2hT = W2[n_neurons:, :].T.astype(jnp.bfloat16)
    gamma2 = gamma.reshape(1, hidden)
    beta2 = beta.reshape(1, hidden)

    grid_a = (n_tokens // TM_A, embed // KT_A)
    h, heads = pl.pallas_call(
        _fc1_ln_kernel,
        grid=grid_a,
        in_specs=[
            pl.BlockSpec((TM_A, KT_A), lambda i, k: (i, k)),
            pl.BlockSpec((KT_A, hidden), lambda i, k: (k, 0)),
            pl.BlockSpec((1, hidden), lambda i, k: (0, 0)),
            pl.BlockSpec((1, hidden), lambda i, k: (0, 0)),
            pl.BlockSpec((hidden, HEADS), lambda i, k: (0, 0)),
        ],
        out_specs=[
            pl.BlockSpec((TM_A, hidden), lambda i, k: (i, 0)),
            pl.BlockSpec((TM_A, HEADS), lambda i, k: (i, 0)),
        ],
        out_shape=[
            jax.ShapeDtypeStruct((n_tokens, hidden), jnp.bfloat16),
            jax.ShapeDtypeStruct((n_tokens, HEADS), jnp.float32),
        ],
        scratch_shapes=[pltpu.VMEM((TM_A, hidden), jnp.float32)],
    )(x, W1T, gamma2, beta2, W2hT)

    grid_b = (n_tokens // TM_B, n_neurons // TN_B)
    neurons = pl.pallas_call(
        _fc2_kernel,
        grid=grid_b,
        in_specs=[
            pl.BlockSpec((TM_B, hidden), lambda i, j: (i, 0)),
            pl.BlockSpec((hidden, TN_B), lambda i, j: (0, j)),
        ],
        out_specs=pl.BlockSpec((TM_B, TN_B), lambda i, j: (i, j)),
        out_shape=jax.ShapeDtypeStruct((n_tokens, n_neurons), jnp.float32),
        compiler_params=pltpu.CompilerParams(
            dimension_semantics=("parallel", "parallel")),
    )(h, W2nT)

    return (neurons, heads)


# speedup vs baseline: 1.0147x; 1.0147x over previous
import jax
import jax.numpy as jnp
from jax.experimental import pallas as pl
from jax.experimental.pallas import tpu as pltpu

HEADS = 32
EPS = 1e-5

TM_A = 512
KT_A = 1024
TM_B = 1024
TN_B = 1024


def _fc1_ln_kernel(x_ref, w1_ref, gamma_ref, beta_ref, w2h_ref,
                   h_ref, heads_ref, acc_ref):
    k = pl.program_id(1)
    nk = pl.num_programs(1)

    d = jnp.dot(x_ref[...].astype(jnp.bfloat16), w1_ref[...],
                preferred_element_type=jnp.float32)

    @pl.when(k == 0)
    def _():
        acc_ref[...] = d

    @pl.when(k != 0)
    def _():
        acc_ref[...] += d

    @pl.when(k == nk - 1)
    def _():
        h = acc_ref[...]
        mu = jnp.mean(h, axis=-1, keepdims=True)
        var = jnp.mean((h - mu) ** 2, axis=-1, keepdims=True)
        hn = (h - mu) * jax.lax.rsqrt(var + EPS) * gamma_ref[...] + beta_ref[...]
        hnb = hn.astype(jnp.bfloat16)
        h_ref[...] = hnb
        heads_ref[...] = jnp.dot(hnb, w2h_ref[...],
                                 preferred_element_type=jnp.float32)


def _fc2_kernel(h_ref, w2n_ref, out_ref):
    out_ref[...] = jax.lax.dot_general(
        h_ref[...], w2n_ref[...],
        (((1,), (1,)), ((), ())),
        preferred_element_type=jnp.float32)


def kernel(x, W1, gamma, beta, W2):
    n_tokens, embed = x.shape
    hidden = W1.shape[0]
    n_out = W2.shape[0]
    n_neurons = n_out - HEADS

    W1T = W1.T.astype(jnp.bfloat16)
    W2n = W2[:n_neurons, :].astype(jnp.bfloat16)
    W2hT = W2[n_neurons:, :].T.astype(jnp.bfloat16)
    gamma2 = gamma.reshape(1, hidden)
    beta2 = beta.reshape(1, hidden)

    grid_a = (n_tokens // TM_A, embed // KT_A)
    h, heads = pl.pallas_call(
        _fc1_ln_kernel,
        grid=grid_a,
        in_specs=[
            pl.BlockSpec((TM_A, KT_A), lambda i, k: (i, k)),
            pl.BlockSpec((KT_A, hidden), lambda i, k: (k, 0)),
            pl.BlockSpec((1, hidden), lambda i, k: (0, 0)),
            pl.BlockSpec((1, hidden), lambda i, k: (0, 0)),
            pl.BlockSpec((hidden, HEADS), lambda i, k: (0, 0)),
        ],
        out_specs=[
            pl.BlockSpec((TM_A, hidden), lambda i, k: (i, 0)),
            pl.BlockSpec((TM_A, HEADS), lambda i, k: (i, 0)),
        ],
        out_shape=[
            jax.ShapeDtypeStruct((n_tokens, hidden), jnp.bfloat16),
            jax.ShapeDtypeStruct((n_tokens, HEADS), jnp.float32),
        ],
        scratch_shapes=[pltpu.VMEM((TM_A, hidden), jnp.float32)],
    )(x, W1T, gamma2, beta2, W2hT)

    grid_b = (n_tokens // TM_B, n_neurons // TN_B)
    neurons = pl.pallas_call(
        _fc2_kernel,
        grid=grid_b,
        in_specs=[
            pl.BlockSpec((TM_B, hidden), lambda i, j: (i, 0)),
            pl.BlockSpec((TN_B, hidden), lambda i, j: (j, 0)),
        ],
        out_specs=pl.BlockSpec((TM_B, TN_B), lambda i, j: (i, j)),
        out_shape=jax.ShapeDtypeStruct((n_tokens, n_neurons), jnp.float32),
        compiler_params=pltpu.CompilerParams(
            dimension_semantics=("parallel", "parallel")),
    )(h, W2n)

    return (neurons, heads)


# speedup vs baseline: 1.0411x; 1.0261x over previous
import jax
import jax.numpy as jnp
from jax.experimental import pallas as pl
from jax.experimental.pallas import tpu as pltpu

HEADS = 32
EPS = 1e-5

TM_A = 256
TM_B = 1024
TN_B = 1024


def _fc1_ln_kernel(x_ref, w1_ref, gamma_ref, beta_ref, w2h_ref,
                   h_ref, heads_ref):
    h = jnp.dot(x_ref[...].astype(jnp.bfloat16), w1_ref[...],
                preferred_element_type=jnp.float32)
    mu = jnp.mean(h, axis=-1, keepdims=True)
    var = jnp.mean((h - mu) ** 2, axis=-1, keepdims=True)
    hn = (h - mu) * jax.lax.rsqrt(var + EPS) * gamma_ref[...] + beta_ref[...]
    hnb = hn.astype(jnp.bfloat16)
    h_ref[...] = hnb
    heads_ref[...] = jnp.dot(hnb, w2h_ref[...],
                             preferred_element_type=jnp.float32)


def _fc2_kernel(h_ref, w2n_ref, out_ref):
    out_ref[...] = jax.lax.dot_general(
        h_ref[...], w2n_ref[...],
        (((1,), (1,)), ((), ())),
        preferred_element_type=jnp.float32)


def kernel(x, W1, gamma, beta, W2):
    n_tokens, embed = x.shape
    hidden = W1.shape[0]
    n_out = W2.shape[0]
    n_neurons = n_out - HEADS

    W1T = W1.T.astype(jnp.bfloat16)
    W2n = W2[:n_neurons, :].astype(jnp.bfloat16)
    W2hT = W2[n_neurons:, :].T.astype(jnp.bfloat16)
    gamma2 = gamma.reshape(1, hidden)
    beta2 = beta.reshape(1, hidden)

    grid_a = (n_tokens // TM_A,)
    h, heads = pl.pallas_call(
        _fc1_ln_kernel,
        grid=grid_a,
        in_specs=[
            pl.BlockSpec((TM_A, embed), lambda i: (i, 0)),
            pl.BlockSpec((embed, hidden), lambda i: (0, 0)),
            pl.BlockSpec((1, hidden), lambda i: (0, 0)),
            pl.BlockSpec((1, hidden), lambda i: (0, 0)),
            pl.BlockSpec((hidden, HEADS), lambda i: (0, 0)),
        ],
        out_specs=[
            pl.BlockSpec((TM_A, hidden), lambda i: (i, 0)),
            pl.BlockSpec((TM_A, HEADS), lambda i: (i, 0)),
        ],
        out_shape=[
            jax.ShapeDtypeStruct((n_tokens, hidden), jnp.bfloat16),
            jax.ShapeDtypeStruct((n_tokens, HEADS), jnp.float32),
        ],
    )(x, W1T, gamma2, beta2, W2hT)

    grid_b = (n_tokens // TM_B, n_neurons // TN_B)
    neurons = pl.pallas_call(
        _fc2_kernel,
        grid=grid_b,
        in_specs=[
            pl.BlockSpec((TM_B, hidden), lambda i, j: (i, 0)),
            pl.BlockSpec((TN_B, hidden), lambda i, j: (j, 0)),
        ],
        out_specs=pl.BlockSpec((TM_B, TN_B), lambda i, j: (i, j)),
        out_shape=jax.ShapeDtypeStruct((n_tokens, n_neurons), jnp.float32),
        compiler_params=pltpu.CompilerParams(
            dimension_semantics=("parallel", "parallel")),
    )(h, W2n)

    return (neurons, heads)
